# baseline (speedup 1.0000x reference)
import jax
import jax.numpy as jnp
from jax import lax
from jax.experimental import pallas as pl
from jax.experimental.pallas import tpu as pltpu
from jax.experimental.pallas import tpu_sc as plsc

N = 10000
E = 320000
IN_DIM = 128
H = 8
D = 16
NEG_SLOPE = 0.2

NC = 2
NS = 16
NW = NC * NS
EW = E // NW
K = 120
NCHUNK = -(-EW // K)
EWP = NCHUNK * K
NACC = N + 8
RPT = N // NS
TW = IN_DIM + 16

TCB = 1000


def _tc_tables_body(h_ref, wsl_ref, wr_ref, tbl_ref, er_ref):
    x = h_ref[...]
    tbl_ref[...] = jnp.dot(x, wsl_ref[...], preferred_element_type=jnp.float32)
    er_ref[...] = jnp.dot(x, wr_ref[...], preferred_element_type=jnp.float32)


def _sc_edge_body(tbl_hbm, er_hbm, src_hbm, dst_hbm, part_hbm,
                  tb0, tb1, eb0, eb1,
                  sv0, sv1, sv2, sv3, dv0, dv1, dv2, dv3,
                  acc,
                  g0, g1, s0, s1, i0, i1, i2, i3):
    c = lax.axis_index("c")
    s = lax.axis_index("s")
    wid = s * NC + c
    tb = [tb0, tb1]
    eb = [eb0, eb1]
    sv = [sv0, sv1, sv2, sv3]
    dv = [dv0, dv1, dv2, dv3]
    gsem = [g0, g1]
    ssem = [s0, s1]
    isem = [i0, i1, i2, i3]

    def idx_issue(t, q):
        pltpu.async_copy(src_hbm.at[wid, t], sv[q], isem[q])
        pltpu.async_copy(dst_hbm.at[wid, t], dv[q], isem[q])

    def idx_wait(q):
        pltpu.make_async_copy(src_hbm.at[wid, 0], sv[q], isem[q]).wait()
        pltpu.make_async_copy(dst_hbm.at[wid, 0], dv[q], isem[q]).wait()

    def gath_issue(q, p):
        pltpu.async_copy(tbl_hbm.at[sv[q]], tb[p], gsem[p])
        pltpu.async_copy(er_hbm.at[dv[q]], eb[p], gsem[p])

    def gath_wait(p):
        pltpu.make_async_copy(tbl_hbm.at[pl.ds(0, K)], tb[p], gsem[p]).wait()
        pltpu.make_async_copy(er_hbm.at[pl.ds(0, K)], eb[p], gsem[p]).wait()

    def scat_issue(p, q):
        pltpu.async_copy(tb[p], acc.at[dv[q]], ssem[p], add=True)

    def scat_wait(p):
        pltpu.make_async_copy(tb[p], acc.at[pl.ds(0, K)], ssem[p]).wait()

    def compute(p):
        tbp = tb[p]
        ebp = eb[p]
        dnums = lax.GatherDimensionNumbers(
            offset_dims=(), collapsed_slice_dims=(0,), start_index_map=(0,))

        @plsc.parallel_loop(0, K, step=1, unroll=4)
        def edge(e):
            el = tbp[e, pl.ds(IN_DIM, 16)]
            er = ebp[e, :]
            x = el + er
            w = jnp.exp(jnp.maximum(x, NEG_SLOPE * x))
            for hh in range(H):
                wh = lax.gather(w, jnp.full((16, 1), hh, jnp.int32), dnums,
                                slice_sizes=(1,),
                                mode=lax.GatherScatterMode.PROMISE_IN_BOUNDS)
                f = tbp[e, pl.ds(hh * 16, 16)]
                tbp[e, pl.ds(hh * 16, 16)] = f * wh
            tbp[e, pl.ds(IN_DIM, 16)] = w

    zeros16 = jnp.zeros((16,), jnp.float32)

    def zrow(i, _):
        for j in range(TW // 16):
            tb0[i, pl.ds(j * 16, 16)] = zeros16
        return 0

    lax.fori_loop(0, K, zrow, 0)
    base = s * RPT
    for q in range(RPT // K):
        pltpu.sync_copy(tb0.at[pl.ds(0, K)], acc.at[pl.ds(base + q * K, K)])
    rem = RPT % K
    if rem:
        pltpu.sync_copy(tb0.at[pl.ds(0, rem)],
                        acc.at[pl.ds(base + (RPT // K) * K, rem)])
    plsc.subcore_barrier()

    def step(t, p2, q_wait, q_issue, q_scat, first=False):
        gath_wait(p2)
        idx_wait(q_wait)
        if not first:
            scat_wait(1 - p2)
        gath_issue(q_wait, 1 - p2)
        idx_issue(jnp.minimum(t + 2, NCHUNK - 1), q_issue)
        compute(p2)
        scat_issue(p2, q_scat)

    idx_issue(0, 0)
    idx_wait(0)
    gath_issue(0, 0)
    idx_issue(1, 1)
    step(jnp.int32(0), 0, 1, 2, 0, first=True)

    def quad(k4, _):
        t = 1 + 4 * k4
        step(t, 1, 2, 3, 1)
        step(t + 1, 0, 3, 0, 2)
        step(t + 2, 1, 0, 1, 3)
        step(t + 3, 0, 1, 2, 0)
        return 0

    n_quads = (NCHUNK - 1) // 4
    lax.fori_loop(0, n_quads, quad, 0)
    for t in range(1 + 4 * n_quads, NCHUNK):
        step(jnp.int32(t), t % 2, (t + 1) % 4, (t + 2) % 4, t % 4)

    gath_wait(NCHUNK % 2)
    idx_wait((NCHUNK + 1) % 4)
    scat_wait((NCHUNK - 1) % 2)

    plsc.subcore_barrier()
    for q in range(RPT // K):
        r0 = base + q * K
        pltpu.sync_copy(acc.at[pl.ds(r0, K)], part_hbm.at[c, pl.ds(r0, K)])
    if rem:
        r0 = base + (RPT // K) * K
        pltpu.sync_copy(acc.at[pl.ds(r0, rem)], part_hbm.at[c, pl.ds(r0, rem)])


def _tc_final_body(h_ref, p0_ref, p1_ref, t_ref, o_ref):
    p = p0_ref[0] + p1_ref[0]
    num = p[:, :IN_DIM]
    den = p[:, IN_DIM:]
    den_exp = jnp.dot(den, t_ref[...], preferred_element_type=jnp.float32)
    r = num / (den_exp + 1e-9)
    o_ref[...] = h_ref[...] + jnp.where(r > 0, r, jnp.exp(r) - 1.0)


@jax.jit
def kernel(h, edge_index, W, attn_l, attn_r):
    f32 = jnp.float32
    W3 = W.reshape(IN_DIM, H, D)
    Wl = (W3 * attn_l[None]).sum(-1)
    Wr = (W3 * attn_r[None]).sum(-1)
    pad = jnp.zeros((IN_DIM, 16 - H), f32)
    Wsl = jnp.concatenate([W, Wl, pad], axis=1)
    Wr16 = jnp.concatenate([Wr, pad], axis=1)

    ei = edge_index.astype(jnp.int32).reshape(2, NW, EW)
    src = jnp.pad(ei[0], ((0, 0), (0, EWP - EW)),
                  constant_values=0).reshape(NW, NCHUNK, K)
    dst = jnp.pad(ei[1], ((0, 0), (0, EWP - EW)),
                  constant_values=N).reshape(NW, NCHUNK, K)

    tbl, er16 = pl.pallas_call(
        _tc_tables_body,
        grid=(N // TCB,),
        in_specs=[
            pl.BlockSpec((TCB, IN_DIM), lambda i: (i, 0)),
            pl.BlockSpec((IN_DIM, TW), lambda i: (0, 0)),
            pl.BlockSpec((IN_DIM, 16), lambda i: (0, 0)),
        ],
        out_specs=[
            pl.BlockSpec((TCB, TW), lambda i: (i, 0)),
            pl.BlockSpec((TCB, 16), lambda i: (i, 0)),
        ],
        out_shape=[
            jax.ShapeDtypeStruct((N, TW), f32),
            jax.ShapeDtypeStruct((NACC, 16), f32),
        ],
    )(h, Wsl, Wr16)

    mesh = plsc.VectorSubcoreMesh(core_axis_name="c", subcore_axis_name="s")
    part = pl.kernel(
        _sc_edge_body,
        out_type=jax.ShapeDtypeStruct((NC, N, TW), f32),
        mesh=mesh,
        scratch_types=(
            [pltpu.VMEM((K, TW), f32)] * 2
            + [pltpu.VMEM((K, 16), f32)] * 2
            + [pltpu.VMEM((K,), jnp.int32)] * 8
            + [pltpu.VMEM_SHARED((NACC, TW), f32)]
            + [pltpu.SemaphoreType.DMA] * 8
        ),
        compiler_params=pltpu.CompilerParams(use_tc_tiling_on_sc=False),
    )(tbl, er16, src, dst)

    T = (jnp.arange(128)[None, :] // D == jnp.arange(16)[:, None]).astype(f32)
    out = pl.pallas_call(
        _tc_final_body,
        grid=(N // TCB,),
        in_specs=[
            pl.BlockSpec((TCB, IN_DIM), lambda i: (i, 0)),
            pl.BlockSpec((1, TCB, TW), lambda i: (0, i, 0)),
            pl.BlockSpec((1, TCB, TW), lambda i: (1, i, 0)),
            pl.BlockSpec((16, IN_DIM), lambda i: (0, 0)),
        ],
        out_specs=pl.BlockSpec((TCB, IN_DIM), lambda i: (i, 0)),
        out_shape=jax.ShapeDtypeStruct((N, IN_DIM), f32),
    )(h, part, part, T)
    return out


# speedup vs baseline: 1.6349x; 1.6349x over previous
import jax
import jax.numpy as jnp
from jax import lax
from jax.experimental import pallas as pl
from jax.experimental.pallas import tpu as pltpu
from jax.experimental.pallas import tpu_sc as plsc

N = 10000
E = 320000
IN_DIM = 128
H = 8
D = 16
NEG_SLOPE = 0.2

NC = 2
NS = 16
NW = NC * NS
EW = E // NW
K = 80
NCHUNK = -(-EW // K)
EWP = NCHUNK * K
NACC = N + 8
RPT = N // NS
TW = IN_DIM + 16

TCB = 1000


def _tc_tables_body(h_ref, wsl_ref, wr_ref, tbl_ref, er_ref):
    x = h_ref[...]
    tbl_ref[...] = jnp.dot(x, wsl_ref[...], preferred_element_type=jnp.float32)
    er_ref[...] = jnp.dot(x, wr_ref[...], preferred_element_type=jnp.float32)


def _sc_edge_body(tbl_hbm, er_hbm, src_hbm, dst_hbm, part_hbm,
                  tb0, tb1, eb0, eb1,
                  sv0, sv1, sv2, sv3, dv0, dv1, dv2, dv3,
                  acc,
                  g0, g1, s0, s1, i0, i1, i2, i3):
    c = lax.axis_index("c")
    s = lax.axis_index("s")
    wid = s * NC + c
    tb = [tb0, tb1]
    eb = [eb0, eb1]
    sv = [sv0, sv1, sv2, sv3]
    dv = [dv0, dv1, dv2, dv3]
    gsem = [g0, g1]
    ssem = [s0, s1]
    isem = [i0, i1, i2, i3]

    def idx_issue(t, q):
        pltpu.async_copy(src_hbm.at[wid, t], sv[q], isem[q])
        pltpu.async_copy(dst_hbm.at[wid, t], dv[q], isem[q])

    def idx_wait(q):
        pltpu.make_async_copy(src_hbm.at[wid, 0], sv[q], isem[q]).wait()
        pltpu.make_async_copy(dst_hbm.at[wid, 0], dv[q], isem[q]).wait()

    def gath_issue(q, p):
        pltpu.async_copy(tbl_hbm.at[sv[q]], tb[p], gsem[p])
        pltpu.async_copy(er_hbm.at[dv[q]], eb[p], gsem[p])

    def gath_wait(p):
        pltpu.make_async_copy(tbl_hbm.at[pl.ds(0, K)], tb[p], gsem[p]).wait()
        pltpu.make_async_copy(er_hbm.at[pl.ds(0, K)], eb[p], gsem[p]).wait()

    def scat_issue(p, q):
        pltpu.async_copy(tb[p], acc.at[dv[q]], ssem[p], add=True)

    def scat_wait(p):
        pltpu.make_async_copy(tb[p], acc.at[pl.ds(0, K)], ssem[p]).wait()

    def compute(p):
        tbp = tb[p]
        ebp = eb[p]
        dnums = lax.GatherDimensionNumbers(
            offset_dims=(), collapsed_slice_dims=(0,), start_index_map=(0,))

        @plsc.parallel_loop(0, K, step=1, unroll=4)
        def edge(e):
            el = tbp[e, pl.ds(IN_DIM, 16)]
            er = ebp[e, :]
            x = el + er
            w = jnp.exp(jnp.maximum(x, NEG_SLOPE * x))
            for hh in range(H):
                wh = lax.gather(w, jnp.full((16, 1), hh, jnp.int32), dnums,
                                slice_sizes=(1,),
                                mode=lax.GatherScatterMode.PROMISE_IN_BOUNDS)
                f = tbp[e, pl.ds(hh * 16, 16)]
                tbp[e, pl.ds(hh * 16, 16)] = f * wh
            tbp[e, pl.ds(IN_DIM, 16)] = w

    zeros16 = jnp.zeros((16,), jnp.float32)

    def zrow(i, _):
        for j in range(TW // 16):
            tb0[i, pl.ds(j * 16, 16)] = zeros16
        return 0

    lax.fori_loop(0, K, zrow, 0)
    base = s * RPT
    for q in range(RPT // K):
        pltpu.sync_copy(tb0.at[pl.ds(0, K)], acc.at[pl.ds(base + q * K, K)])
    rem = RPT % K
    if rem:
        pltpu.sync_copy(tb0.at[pl.ds(0, rem)],
                        acc.at[pl.ds(base + (RPT // K) * K, rem)])
    plsc.subcore_barrier()

    def step(t, p2, q_wait, q_issue, q_scat, first=False):
        gath_wait(p2)
        idx_wait(q_wait)
        if not first:
            scat_wait(1 - p2)
        gath_issue(q_wait, 1 - p2)
        idx_issue(jnp.minimum(t + 2, NCHUNK - 1), q_issue)
        compute(p2)
        scat_issue(p2, q_scat)

    idx_issue(0, 0)
    idx_wait(0)
    gath_issue(0, 0)
    idx_issue(1, 1)
    step(jnp.int32(0), 0, 1, 2, 0, first=True)

    def quad(k4, _):
        t = 1 + 4 * k4
        step(t, 1, 2, 3, 1)
        step(t + 1, 0, 3, 0, 2)
        step(t + 2, 1, 0, 1, 3)
        step(t + 3, 0, 1, 2, 0)
        return 0

    n_quads = (NCHUNK - 1) // 4
    lax.fori_loop(0, n_quads, quad, 0)
    for t in range(1 + 4 * n_quads, NCHUNK):
        step(jnp.int32(t), t % 2, (t + 1) % 4, (t + 2) % 4, t % 4)

    gath_wait(NCHUNK % 2)
    idx_wait((NCHUNK + 1) % 4)
    scat_wait((NCHUNK - 1) % 2)

    plsc.subcore_barrier()
    for q in range(RPT // K):
        r0 = base + q * K
        pltpu.sync_copy(acc.at[pl.ds(r0, K)], part_hbm.at[c, pl.ds(r0, K)])
    if rem:
        r0 = base + (RPT // K) * K
        pltpu.sync_copy(acc.at[pl.ds(r0, rem)], part_hbm.at[c, pl.ds(r0, rem)])


def _tc_final_body(h_ref, p0_ref, p1_ref, t_ref, o_ref):
    p = p0_ref[0] + p1_ref[0]
    num = p[:, :IN_DIM]
    den = p[:, IN_DIM:]
    den_exp = jnp.dot(den, t_ref[...], preferred_element_type=jnp.float32)
    r = num / (den_exp + 1e-9)
    o_ref[...] = h_ref[...] + jnp.where(r > 0, r, jnp.exp(r) - 1.0)


@jax.jit
def kernel(h, edge_index, W, attn_l, attn_r):
    f32 = jnp.float32
    W3 = W.reshape(IN_DIM, H, D)
    Wl = (W3 * attn_l[None]).sum(-1)
    Wr = (W3 * attn_r[None]).sum(-1)
    pad = jnp.zeros((IN_DIM, 16 - H), f32)
    Wsl = jnp.concatenate([W, Wl, pad], axis=1)
    Wr16 = jnp.concatenate([Wr, pad], axis=1)

    ei = edge_index.astype(jnp.int32).reshape(2, NW, EW)
    src = jnp.pad(ei[0], ((0, 0), (0, EWP - EW)),
                  constant_values=0).reshape(NW, NCHUNK, K)
    dst = jnp.pad(ei[1], ((0, 0), (0, EWP - EW)),
                  constant_values=N).reshape(NW, NCHUNK, K)

    tbl, er16 = pl.pallas_call(
        _tc_tables_body,
        grid=(N // TCB,),
        in_specs=[
            pl.BlockSpec((TCB, IN_DIM), lambda i: (i, 0)),
            pl.BlockSpec((IN_DIM, TW), lambda i: (0, 0)),
            pl.BlockSpec((IN_DIM, 16), lambda i: (0, 0)),
        ],
        out_specs=[
            pl.BlockSpec((TCB, TW), lambda i: (i, 0)),
            pl.BlockSpec((TCB, 16), lambda i: (i, 0)),
        ],
        out_shape=[
            jax.ShapeDtypeStruct((N, TW), f32),
            jax.ShapeDtypeStruct((NACC, 16), f32),
        ],
    )(h, Wsl, Wr16)

    mesh = plsc.VectorSubcoreMesh(core_axis_name="c", subcore_axis_name="s")
    part = pl.kernel(
        _sc_edge_body,
        out_type=jax.ShapeDtypeStruct((NC, N, TW), f32),
        mesh=mesh,
        scratch_types=(
            [pltpu.VMEM((K, TW), f32)] * 2
            + [pltpu.VMEM((K, 16), f32)] * 2
            + [pltpu.VMEM((K,), jnp.int32)] * 8
            + [pltpu.VMEM_SHARED((NACC, TW), f32)]
            + [pltpu.SemaphoreType.DMA] * 8
        ),
        compiler_params=pltpu.CompilerParams(use_tc_tiling_on_sc=False),
    )(tbl, er16, src, dst)

    T = (jnp.arange(128)[None, :] // D == jnp.arange(16)[:, None]).astype(f32)
    out = pl.pallas_call(
        _tc_final_body,
        grid=(N // TCB,),
        in_specs=[
            pl.BlockSpec((TCB, IN_DIM), lambda i: (i, 0)),
            pl.BlockSpec((1, TCB, TW), lambda i: (0, i, 0)),
            pl.BlockSpec((1, TCB, TW), lambda i: (1, i, 0)),
            pl.BlockSpec((16, IN_DIM), lambda i: (0, 0)),
        ],
        out_specs=pl.BlockSpec((TCB, IN_DIM), lambda i: (i, 0)),
        out_shape=jax.ShapeDtypeStruct((N, IN_DIM), f32),
    )(h, part, part, T)
    return out


# speedup vs baseline: 1.8795x; 1.1496x over previous
import jax
import jax.numpy as jnp
from jax import lax
from jax.experimental import pallas as pl
from jax.experimental.pallas import tpu as pltpu
from jax.experimental.pallas import tpu_sc as plsc

N = 10000
E = 320000
IN_DIM = 128
H = 8
D = 16
NEG_SLOPE = 0.2

NC = 2
NS = 16
NW = NC * NS
EW = E // NW
K = 80
NCHUNK = -(-EW // K)
EWP = NCHUNK * K
NACC = N + 8
RPT = N // NS
TW = IN_DIM + 16

TCB = 1000


def _tc_tables_body(h_ref, wsl_ref, wr_ref, tbl_ref, er_ref):
    x = h_ref[...]
    tbl_ref[...] = jnp.dot(x, wsl_ref[...], preferred_element_type=jnp.float32)
    er_ref[...] = jnp.dot(x, wr_ref[...], preferred_element_type=jnp.float32)


def _sc_edge_body(tbl_hbm, er_hbm, src_hbm, dst_hbm, part_hbm,
                  tb0, tb1, tb2, eb0, eb1, eb2,
                  sv0, sv1, sv2, sv3, sv4, sv5,
                  dv0, dv1, dv2, dv3, dv4, dv5,
                  acc,
                  g0, g1, g2, s0, s1, s2,
                  i0, i1, i2, i3, i4, i5):
    c = lax.axis_index("c")
    s = lax.axis_index("s")
    wid = s * NC + c
    tb = [tb0, tb1, tb2]
    eb = [eb0, eb1, eb2]
    sv = [sv0, sv1, sv2, sv3, sv4, sv5]
    dv = [dv0, dv1, dv2, dv3, dv4, dv5]
    gsem = [g0, g1, g2]
    ssem = [s0, s1, s2]
    isem = [i0, i1, i2, i3, i4, i5]

    def idx_issue(t, q):
        pltpu.async_copy(src_hbm.at[wid, t], sv[q], isem[q])
        pltpu.async_copy(dst_hbm.at[wid, t], dv[q], isem[q])

    def idx_wait(q):
        pltpu.make_async_copy(src_hbm.at[wid, 0], sv[q], isem[q]).wait()
        pltpu.make_async_copy(dst_hbm.at[wid, 0], dv[q], isem[q]).wait()

    def gath_issue(q, p):
        pltpu.async_copy(tbl_hbm.at[sv[q]], tb[p], gsem[p])
        pltpu.async_copy(er_hbm.at[dv[q]], eb[p], gsem[p])

    def gath_wait(p):
        pltpu.make_async_copy(tbl_hbm.at[pl.ds(0, K)], tb[p], gsem[p]).wait()
        pltpu.make_async_copy(er_hbm.at[pl.ds(0, K)], eb[p], gsem[p]).wait()

    def scat_issue(p, q):
        pltpu.async_copy(tb[p], acc.at[dv[q]], ssem[p], add=True)

    def scat_wait(p):
        pltpu.make_async_copy(tb[p], acc.at[pl.ds(0, K)], ssem[p]).wait()

    def compute(p):
        tbp = tb[p]
        ebp = eb[p]
        dnums = lax.GatherDimensionNumbers(
            offset_dims=(), collapsed_slice_dims=(0,), start_index_map=(0,))

        @plsc.parallel_loop(0, K, step=1, unroll=4)
        def edge(e):
            el = tbp[e, pl.ds(IN_DIM, 16)]
            er = ebp[e, :]
            x = el + er
            w = jnp.exp(jnp.maximum(x, NEG_SLOPE * x))
            for hh in range(H):
                wh = lax.gather(w, jnp.full((16, 1), hh, jnp.int32), dnums,
                                slice_sizes=(1,),
                                mode=lax.GatherScatterMode.PROMISE_IN_BOUNDS)
                f = tbp[e, pl.ds(hh * 16, 16)]
                tbp[e, pl.ds(hh * 16, 16)] = f * wh
            tbp[e, pl.ds(IN_DIM, 16)] = w

    zeros16 = jnp.zeros((16,), jnp.float32)

    def zrow(i, _):
        for j in range(TW // 16):
            tb0[i, pl.ds(j * 16, 16)] = zeros16
        return 0

    lax.fori_loop(0, K, zrow, 0)
    base = s * RPT
    for q in range(RPT // K):
        pltpu.sync_copy(tb0.at[pl.ds(0, K)], acc.at[pl.ds(base + q * K, K)])
    rem = RPT % K
    if rem:
        pltpu.sync_copy(tb0.at[pl.ds(0, rem)],
                        acc.at[pl.ds(base + (RPT // K) * K, rem)])
    plsc.subcore_barrier()

    def step(t, p, s_cur, s_g2, s_i3, first=False):
        gath_wait(p)
        idx_wait(s_g2)
        if not first:
            scat_wait((p + 2) % 3)
        gath_issue(s_g2, (p + 2) % 3)
        idx_issue(jnp.minimum(t + 3, NCHUNK - 1), s_i3)
        compute(p)
        scat_issue(p, s_cur)

    idx_issue(0, 0)
    idx_issue(1, 1)
    idx_issue(2, 2)
    idx_wait(0)
    gath_issue(0, 0)
    idx_wait(1)
    gath_issue(1, 1)
    step(jnp.int32(0), 0, 0, 2, 3, first=True)

    def hexa(k6, _):
        t = 1 + 6 * k6
        for j in range(6):
            tj = 1 + j
            step(t + j, tj % 3, tj % 6, (tj + 2) % 6, (tj + 3) % 6)
        return 0

    n_hex = (NCHUNK - 1) // 6
    lax.fori_loop(0, n_hex, hexa, 0)
    for t in range(1 + 6 * n_hex, NCHUNK):
        step(jnp.int32(t), t % 3, t % 6, (t + 2) % 6, (t + 3) % 6)

    gath_wait(NCHUNK % 3)
    gath_wait((NCHUNK + 1) % 3)
    idx_wait((NCHUNK + 2) % 6)
    scat_wait((NCHUNK - 1) % 3)

    plsc.subcore_barrier()
    for q in range(RPT // K):
        r0 = base + q * K
        pltpu.sync_copy(acc.at[pl.ds(r0, K)], part_hbm.at[c, pl.ds(r0, K)])
    if rem:
        r0 = base + (RPT // K) * K
        pltpu.sync_copy(acc.at[pl.ds(r0, rem)], part_hbm.at[c, pl.ds(r0, rem)])


def _tc_final_body(h_ref, p0_ref, p1_ref, t_ref, o_ref):
    p = p0_ref[0] + p1_ref[0]
    num = p[:, :IN_DIM]
    den = p[:, IN_DIM:]
    den_exp = jnp.dot(den, t_ref[...], preferred_element_type=jnp.float32)
    r = num / (den_exp + 1e-9)
    o_ref[...] = h_ref[...] + jnp.where(r > 0, r, jnp.exp(r) - 1.0)


@jax.jit
def kernel(h, edge_index, W, attn_l, attn_r):
    f32 = jnp.float32
    W3 = W.reshape(IN_DIM, H, D)
    Wl = (W3 * attn_l[None]).sum(-1)
    Wr = (W3 * attn_r[None]).sum(-1)
    pad = jnp.zeros((IN_DIM, 16 - H), f32)
    Wsl = jnp.concatenate([W, Wl, pad], axis=1)
    Wr16 = jnp.concatenate([Wr, pad], axis=1)

    ei = edge_index.astype(jnp.int32).reshape(2, NW, EW)
    src = jnp.pad(ei[0], ((0, 0), (0, EWP - EW)),
                  constant_values=0).reshape(NW, NCHUNK, K)
    dst = jnp.pad(ei[1], ((0, 0), (0, EWP - EW)),
                  constant_values=N).reshape(NW, NCHUNK, K)

    tbl, er16 = pl.pallas_call(
        _tc_tables_body,
        grid=(N // TCB,),
        in_specs=[
            pl.BlockSpec((TCB, IN_DIM), lambda i: (i, 0)),
            pl.BlockSpec((IN_DIM, TW), lambda i: (0, 0)),
            pl.BlockSpec((IN_DIM, 16), lambda i: (0, 0)),
        ],
        out_specs=[
            pl.BlockSpec((TCB, TW), lambda i: (i, 0)),
            pl.BlockSpec((TCB, 16), lambda i: (i, 0)),
        ],
        out_shape=[
            jax.ShapeDtypeStruct((N, TW), f32),
            jax.ShapeDtypeStruct((NACC, 16), f32),
        ],
    )(h, Wsl, Wr16)

    mesh = plsc.VectorSubcoreMesh(core_axis_name="c", subcore_axis_name="s")
    part = pl.kernel(
        _sc_edge_body,
        out_type=jax.ShapeDtypeStruct((NC, N, TW), f32),
        mesh=mesh,
        scratch_types=(
            [pltpu.VMEM((K, TW), f32)] * 3
            + [pltpu.VMEM((K, 16), f32)] * 3
            + [pltpu.VMEM((K,), jnp.int32)] * 12
            + [pltpu.VMEM_SHARED((NACC, TW), f32)]
            + [pltpu.SemaphoreType.DMA] * 12
        ),
        compiler_params=pltpu.CompilerParams(use_tc_tiling_on_sc=False),
    )(tbl, er16, src, dst)

    T = (jnp.arange(128)[None, :] // D == jnp.arange(16)[:, None]).astype(f32)
    out = pl.pallas_call(
        _tc_final_body,
        grid=(N // TCB,),
        in_specs=[
            pl.BlockSpec((TCB, IN_DIM), lambda i: (i, 0)),
            pl.BlockSpec((1, TCB, TW), lambda i: (0, i, 0)),
            pl.BlockSpec((1, TCB, TW), lambda i: (1, i, 0)),
            pl.BlockSpec((16, IN_DIM), lambda i: (0, 0)),
        ],
        out_specs=pl.BlockSpec((TCB, IN_DIM), lambda i: (i, 0)),
        out_shape=jax.ShapeDtypeStruct((N, IN_DIM), f32),
    )(h, part, part, T)
    return out
